# baseline (speedup 1.0000x reference)
import functools

import jax
import jax.numpy as jnp
from jax.experimental import pallas as pl

N = 10000
NFEAT = 128
NCLASSES = 16

ROW_BLK = 400


def _support1_kernel(x_ref, w1_ref, out_ref):
    out_ref[...] = jnp.dot(x_ref[...], w1_ref[...],
                           preferred_element_type=jnp.float32)


def _pass2_kernel(adj_ref, s1_ref, b1_ref, w2_ref, s2_ref):
    acc = jnp.dot(adj_ref[...], s1_ref[...],
                  preferred_element_type=jnp.float32)
    feat = jnp.maximum(acc + b1_ref[...], 0.0)
    s2_ref[...] = jnp.dot(feat, w2_ref[...],
                          preferred_element_type=jnp.float32)


def _pass3_kernel(adj_ref, s2_ref, b2_ref, out_ref):
    out_ref[...] = jnp.dot(adj_ref[...], s2_ref[...],
                           preferred_element_type=jnp.float32) + b2_ref[...]


@jax.jit
def kernel(x, adj, W1, b1, W2, b2):
    b1r = b1.reshape(1, NFEAT)
    b2r = b2.reshape(1, NCLASSES)

    support1 = pl.pallas_call(
        _support1_kernel,
        grid=(5,),
        in_specs=[
            pl.BlockSpec((N // 5, NFEAT), lambda i: (i, 0)),
            pl.BlockSpec((NFEAT, NFEAT), lambda i: (0, 0)),
        ],
        out_specs=pl.BlockSpec((N // 5, NFEAT), lambda i: (i, 0)),
        out_shape=jax.ShapeDtypeStruct((N, NFEAT), jnp.float32),
    )(x, W1)

    nblk = N // ROW_BLK
    support2 = pl.pallas_call(
        _pass2_kernel,
        grid=(nblk,),
        in_specs=[
            pl.BlockSpec((ROW_BLK, N), lambda i: (i, 0)),
            pl.BlockSpec((N, NFEAT), lambda i: (0, 0)),
            pl.BlockSpec((1, NFEAT), lambda i: (0, 0)),
            pl.BlockSpec((NFEAT, NCLASSES), lambda i: (0, 0)),
        ],
        out_specs=pl.BlockSpec((ROW_BLK, NCLASSES), lambda i: (i, 0)),
        out_shape=jax.ShapeDtypeStruct((N, NCLASSES), jnp.float32),
    )(adj, support1, b1r, W2)

    logit = pl.pallas_call(
        _pass3_kernel,
        grid=(nblk,),
        in_specs=[
            pl.BlockSpec((ROW_BLK, N), lambda i: (i, 0)),
            pl.BlockSpec((N, NCLASSES), lambda i: (0, 0)),
            pl.BlockSpec((1, NCLASSES), lambda i: (0, 0)),
        ],
        out_specs=pl.BlockSpec((ROW_BLK, NCLASSES), lambda i: (i, 0)),
        out_shape=jax.ShapeDtypeStruct((N, NCLASSES), jnp.float32),
    )(adj, support2, b2r)

    return logit
